# baseline (speedup 1.0000x reference)
import functools

import jax
import jax.numpy as jnp
import numpy as np
from jax import lax
from jax.experimental import pallas as pl
from jax.experimental.pallas import tpu as pltpu
from jax.experimental.pallas import tpu_sc as plsc

B, N = 4, 4096
SEM_IN, SEM_OUT = 128, 13
INS_IN, INS_OUT = 128, 32
K = 32

NBLK = 512
DRB = 256
NBM = 32
NCH = 256

NW = 32
ROWS_W = (B * N) // NW
G = 4
NGRP = ROWS_W // (2 * G)
CAP = N


def _dense_front_body(fsem_ref, fins_ref, Wad_ref, bad_ref, Wins_ref, bins_ref,
                      eins_ref):
    fsem = fsem_ref[0]
    fins = fins_ref[0]
    adapted = jnp.maximum(
        jax.lax.dot_general(Wad_ref[...], fsem, (((1,), (0,)), ((), ())),
                            preferred_element_type=jnp.float32) + bad_ref[...],
        0.0)
    f_sins = fins + adapted
    eins = jax.lax.dot_general(Wins_ref[...], f_sins, (((1,), (0,)), ((), ())),
                               preferred_element_type=jnp.float32) + bins_ref[...]
    eins_ref[0] = eins


def _dense_front(f_sem, f_ins, W_eff, b_eff, W_ins, b_ins):
    return pl.pallas_call(
        _dense_front_body,
        grid=(B, N // NBLK),
        in_specs=[
            pl.BlockSpec((1, SEM_IN, NBLK), lambda b, n: (b, 0, n)),
            pl.BlockSpec((1, INS_IN, NBLK), lambda b, n: (b, 0, n)),
            pl.BlockSpec((INS_IN, SEM_IN), lambda b, n: (0, 0)),
            pl.BlockSpec((INS_IN, 1), lambda b, n: (0, 0)),
            pl.BlockSpec((INS_OUT, INS_IN), lambda b, n: (0, 0)),
            pl.BlockSpec((INS_OUT, 1), lambda b, n: (0, 0)),
        ],
        out_specs=pl.BlockSpec((1, INS_OUT, NBLK), lambda b, n: (b, 0, n)),
        out_shape=jax.ShapeDtypeStruct((B, INS_OUT, N), jnp.float32),
    )(f_sem, f_ins, W_eff, b_eff, W_ins, b_ins)


def _dist_body(p_ref, e_ref, d_ref, bm_ref, cnt_ref):
    p = p_ref[0]
    e = e_ref[0]
    sq_r = jnp.sum(p * p, axis=1, keepdims=True)
    sq_c = jnp.sum(e * e, axis=0, keepdims=True)
    dot = jax.lax.dot_general(p, e, (((1,), (0,)), ((), ())),
                              preferred_element_type=jnp.float32)
    d = (sq_r - 2.0 * dot) + sq_c
    d_ref[0] = d
    bm = jnp.min(d.reshape(DRB, NBM, N // NBM), axis=-1)
    t = jnp.max(bm, axis=-1, keepdims=True)
    bm_ref[0] = jnp.broadcast_to(t, (DRB, NBM))
    m01 = jnp.where(d <= t, 1.0, 0.0)
    cnt_ref[0] = jnp.sum(m01.reshape(DRB, NCH, 16), axis=-1)


def _dist(p, e_ins):
    return pl.pallas_call(
        _dist_body,
        grid=(B, N // DRB),
        in_specs=[
            pl.BlockSpec((1, DRB, INS_OUT), lambda b, n: (b, n, 0)),
            pl.BlockSpec((1, INS_OUT, N), lambda b, n: (b, 0, 0)),
        ],
        out_specs=[
            pl.BlockSpec((1, DRB, N), lambda b, n: (b, n, 0)),
            pl.BlockSpec((1, DRB, NBM), lambda b, n: (b, n, 0)),
            pl.BlockSpec((1, DRB, NCH), lambda b, n: (b, n, 0)),
        ],
        out_shape=[
            jax.ShapeDtypeStruct((B, N, N), jnp.float32),
            jax.ShapeDtypeStruct((B, N, NBM), jnp.float32),
            jax.ShapeDtypeStruct((B, N, NCH), jnp.float32),
        ],
    )(p, e_ins)


def _gperm(x, idx):
    dnums = lax.GatherDimensionNumbers(
        offset_dims=(), collapsed_slice_dims=(0,), start_index_map=(0,))
    return lax.gather(x, idx[:, None], dnums, (1,),
                      mode=lax.GatherScatterMode.PROMISE_IN_BOUNDS)


def _tree_min(v, perms):
    for p in perms:
        v = jnp.minimum(v, _gperm(v, p))
    return v


def _tree_sum(v, perms):
    for p in perms:
        v = v + _gperm(v, p)
    return v


def _cmpx(kv, ki, pv, pi, keepmin):
    lt = kv < pv
    mn = jnp.minimum(kv, pv)
    mx = jnp.maximum(kv, pv)
    mni = jnp.where(lt, ki, pi)
    mxi = jnp.where(lt, pi, ki)
    return jnp.where(keepmin, mn, mx), jnp.where(keepmin, mni, mxi)


def _km(it, ksz, j):
    partner = it ^ j
    ltp = jnp.where(it < partner, 1, 0)
    if ksz >= 16:
        return ltp > 0
    up = jnp.where((it & ksz) == 0, 1, 0)
    return ltp == up


def _sort16(kv, ki, it):
    for ksz in (2, 4, 8, 16):
        j = ksz >> 1
        while j >= 1:
            perm = it ^ j
            keepmin = _km(it, ksz, j)
            pk = _gperm(kv, perm)
            pi = _gperm(ki, perm)
            kv, ki = _cmpx(kv, ki, pk, pi, keepmin)
            j >>= 1
    return kv, ki


def _clean16(kv, ki, it):
    for j in (8, 4, 2, 1):
        perm = it ^ j
        keepmin = _km(it, 16, j)
        pk = _gperm(kv, perm)
        pi = _gperm(ki, perm)
        kv, ki = _cmpx(kv, ki, pk, pi, keepmin)
    return kv, ki


def _merge32_low(alov, aloi, ahiv, ahii, bv, bi, it):
    rbv = lax.rev(bv, (0,))
    rbi = lax.rev(bi, (0,))
    lt = ahiv < rbv
    l1v = jnp.minimum(ahiv, rbv)
    l1i = jnp.where(lt, ahii, rbi)
    lt2 = alov < l1v
    lov = jnp.minimum(alov, l1v)
    hiv = jnp.maximum(alov, l1v)
    loi = jnp.where(lt2, aloi, l1i)
    hii = jnp.where(lt2, l1i, aloi)
    lov, loi = _clean16(lov, loi, it)
    hiv, hii = _clean16(hiv, hii, it)
    return lov, loi, hiv, hii


def _sc_knn_pool(d_hbm, bm_hbm, cnt_hbm, fsemT_hbm, out_hbm,
                 row_buf, bm_buf, cnt_buf, cand_val, cand_idx, idx_buf, gbuf,
                 ostage, dsem0, dsem1, gsem0, gsem1):
    nc = 2
    wid = lax.axis_index("s") * nc + lax.axis_index("c")
    r0 = wid * ROWS_W
    boff = (wid // 8) * N
    it = lax.iota(jnp.int32, 16)
    inf = jnp.full((16,), jnp.inf, jnp.float32)
    zero_i = jnp.zeros((16,), jnp.int32)
    perms = tuple(it ^ sh for sh in (1, 2, 4, 8))
    dsems = (dsem0, dsem1)
    gsems = (gsem0, gsem1)

    def d_dma(slot, grp):
        base = r0 + grp * G
        return pltpu.make_async_copy(d_hbm.at[pl.ds(base * N, G * N)],
                                     row_buf.at[slot], dsems[slot])

    def bm_dma(slot, grp):
        base = r0 + grp * G
        return pltpu.make_async_copy(bm_hbm.at[pl.ds(base * NBM, G * NBM)],
                                     bm_buf.at[slot], dsems[slot])

    def cnt_dma(slot, grp):
        base = r0 + grp * G
        return pltpu.make_async_copy(cnt_hbm.at[pl.ds(base * NCH, G * NCH)],
                                     cnt_buf.at[slot], dsems[slot])

    def g_dma(slot, kk):
        return pltpu.make_async_copy(
            fsemT_hbm.at[idx_buf.at[slot, pl.ds(kk * K, K)]],
            gbuf.at[slot, pl.ds(kk * K, K)], gsems[slot])

    for s in (0, 1):
        d_dma(s, s).start()
        bm_dma(s, s).start()
        cnt_dma(s, s).start()

    def process_rows(s):
        def row_body(kk, _):
            t_spl = bm_buf[s, pl.ds(kk * NBM, 16)]
            del t_spl

            def scan_group(jg, off):
                cv = cnt_buf[s, pl.ds(kk * NCH + jg * 16, 16)]
                for L in range(16):
                    cnt = lax.convert_element_type(cv[L], jnp.int32)
                    jbase = jnp.full((16,), (jg * 16 + L) * 16, jnp.int32)
                    v0 = row_buf[s, pl.ds(kk * N + (jg * 16 + L) * 16, 16)]

                    def extract_body(_, carry, jbase=jbase):
                        vv, ofs = carry
                        mn = _tree_min(vv, perms)
                        eq = vv == mn
                        lanev = jnp.where(eq, it, 16)
                        lane = _tree_min(lanev, perms)
                        cand_val[pl.ds(ofs, 16)] = mn
                        cand_idx[pl.ds(ofs, 16)] = lane + jbase
                        vv = jnp.where(it == lane, jnp.inf, vv)
                        return (vv, ofs + 1)

                    _, off = lax.fori_loop(0, cnt, extract_body, (v0, off))
                return off

            c = lax.fori_loop(0, NCH // 16, scan_group, 0)

            c_spl = jnp.full((16,), c, jnp.int32)
            nch = (c + 15) // 16

            def sel_body(i, carry):
                alov, aloi, ahiv, ahii = carry
                cv = cand_val[pl.ds(i * 16, 16)]
                ci = cand_idx[pl.ds(i * 16, 16)]
                valid = (it + jnp.full((16,), i * 16, jnp.int32)) < c_spl
                cv = jnp.where(valid, cv, jnp.inf)
                sv, si = _sort16(cv, ci, it)
                return _merge32_low(alov, aloi, ahiv, ahii, sv, si, it)

            _, loi, _, hii = lax.fori_loop(0, nch, sel_body,
                                           (inf, zero_i, inf, zero_i))
            bo = jnp.full((16,), boff, jnp.int32)
            idx_buf[s, pl.ds(kk * K, 16)] = loi + bo
            idx_buf[s, pl.ds(kk * K + 16, 16)] = hii + bo
            g_dma(s, kk).start()
            return 0

        lax.fori_loop(0, G, row_body, 0)

    def max_rows(s, grp):
        def max_body(kk, _):
            g_dma(s, kk).wait()
            for cc in range(8):
                acc = gbuf[s, pl.ds(kk * K, 1), pl.ds(16 * cc, 16)][0]
                for r in range(1, K):
                    acc = jnp.maximum(
                        acc, gbuf[s, pl.ds(kk * K + r, 1), pl.ds(16 * cc, 16)][0])
                ostage[pl.ds(kk * SEM_IN + 16 * cc, 16)] = acc
            return 0

        lax.fori_loop(0, G, max_body, 0)
        base = r0 + grp * G
        pltpu.sync_copy(ostage, out_hbm.at[pl.ds(base * SEM_IN, G * SEM_IN)])

    def body(gi, _):
        for s in (0, 1):
            grp = 2 * gi + s
            d_dma(s, grp).wait()
            bm_dma(s, grp).wait()
            cnt_dma(s, grp).wait()
            process_rows(s)
            nxt = jnp.minimum(grp + 2, ROWS_W // G - 1)
            base_n = r0 + nxt * G
            pltpu.make_async_copy(d_hbm.at[pl.ds(base_n * N, G * N)],
                                  row_buf.at[s], dsems[s]).start()
            pltpu.make_async_copy(bm_hbm.at[pl.ds(base_n * NBM, G * NBM)],
                                  bm_buf.at[s], dsems[s]).start()
            pltpu.make_async_copy(cnt_hbm.at[pl.ds(base_n * NCH, G * NCH)],
                                  cnt_buf.at[s], dsems[s]).start()

            @pl.when(grp > 0)
            def _():
                max_rows(1 - s, grp - 1)
        return 0

    lax.fori_loop(0, NGRP, body, 0)
    max_rows(1, ROWS_W // G - 1)
    for s in (0, 1):
        d_dma(s, 0).wait()
        bm_dma(s, 0).wait()
        cnt_dma(s, 0).wait()


def _sc_knn(d2, bm2, cnt2, fsemT):
    mesh = plsc.VectorSubcoreMesh(core_axis_name="c", subcore_axis_name="s")
    call = pl.kernel(
        _sc_knn_pool, mesh=mesh,
        out_type=jax.ShapeDtypeStruct((B * N * SEM_IN,), jnp.float32),
        scratch_types=[
            pltpu.VMEM((2, G * N), jnp.float32),
            pltpu.VMEM((2, G * NBM), jnp.float32),
            pltpu.VMEM((2, G * NCH), jnp.float32),
            pltpu.VMEM((CAP + 16,), jnp.float32),
            pltpu.VMEM((CAP + 16,), jnp.int32),
            pltpu.VMEM((2, G * K), jnp.int32),
            pltpu.VMEM((2, G * K, SEM_IN), jnp.float32),
            pltpu.VMEM((G * SEM_IN,), jnp.float32),
            pltpu.SemaphoreType.DMA,
            pltpu.SemaphoreType.DMA,
            pltpu.SemaphoreType.DMA,
            pltpu.SemaphoreType.DMA,
        ],
    )
    return call(d2.reshape(-1), bm2.reshape(-1), cnt2.reshape(-1),
                fsemT).reshape(B * N, SEM_IN)


def _proj_body(fisem_ref, W_ref, b_ref, out_ref):
    out_ref[...] = jax.lax.dot_general(
        fisem_ref[...], W_ref[...], (((1,), (0,)), ((), ())),
        preferred_element_type=jnp.float32) + b_ref[...]


def _proj(f_isemT, W_semT, b_sem):
    return pl.pallas_call(
        _proj_body,
        grid=((B * N) // NBLK,),
        in_specs=[
            pl.BlockSpec((NBLK, SEM_IN), lambda n: (n, 0)),
            pl.BlockSpec((SEM_IN, SEM_OUT), lambda n: (0, 0)),
            pl.BlockSpec((1, SEM_OUT), lambda n: (0, 0)),
        ],
        out_specs=pl.BlockSpec((NBLK, SEM_OUT), lambda n: (n, 0)),
        out_shape=jax.ShapeDtypeStruct((B * N, SEM_OUT), jnp.float32),
    )(f_isemT, W_semT, b_sem)


def kernel(f_sem, f_ins, W_ad, b_ad, gamma_ad, beta_ad, W_ins, b_ins, W_sem, b_sem):
    W_eff = gamma_ad[:, None] * W_ad
    b_eff = (gamma_ad * b_ad + beta_ad)[:, None]
    e_ins = _dense_front(f_sem, f_ins, W_eff, b_eff, W_ins, b_ins[:, None])

    p = jnp.transpose(e_ins, (0, 2, 1))
    d, bm, cnt = _dist(p, e_ins)

    fsemT = jnp.transpose(f_sem, (0, 2, 1)).reshape(B * N, SEM_IN)
    f_isemT = _sc_knn(d.reshape(B * N, N), bm.reshape(B * N, NBM),
                      cnt.reshape(B * N, NCH), fsemT)

    p_semT = _proj(f_isemT, jnp.transpose(W_sem), b_sem[None, :])
    p_sem = jnp.transpose(p_semT.reshape(B, N, SEM_OUT), (0, 2, 1))
    return (p_sem, e_ins)


# speedup vs baseline: 1.1359x; 1.1359x over previous
import functools

import jax
import jax.numpy as jnp
import numpy as np
from jax import lax
from jax.experimental import pallas as pl
from jax.experimental.pallas import tpu as pltpu
from jax.experimental.pallas import tpu_sc as plsc

B, N = 4, 4096
SEM_IN, SEM_OUT = 128, 13
INS_IN, INS_OUT = 128, 32
K = 32

NBLK = 512
DRB = 256
NBM = 32
NCH = 256

NW = 32
ROWS_W = (B * N) // NW
G = 2
NGRP = ROWS_W // (2 * G)
CAP = N


def _dense_front_body(fsem_ref, fins_ref, Wad_ref, bad_ref, Wins_ref, bins_ref,
                      eins_ref):
    fsem = fsem_ref[0]
    fins = fins_ref[0]
    adapted = jnp.maximum(
        jax.lax.dot_general(Wad_ref[...], fsem, (((1,), (0,)), ((), ())),
                            preferred_element_type=jnp.float32) + bad_ref[...],
        0.0)
    f_sins = fins + adapted
    eins = jax.lax.dot_general(Wins_ref[...], f_sins, (((1,), (0,)), ((), ())),
                               preferred_element_type=jnp.float32) + bins_ref[...]
    eins_ref[0] = eins


def _dense_front(f_sem, f_ins, W_eff, b_eff, W_ins, b_ins):
    return pl.pallas_call(
        _dense_front_body,
        grid=(B, N // NBLK),
        in_specs=[
            pl.BlockSpec((1, SEM_IN, NBLK), lambda b, n: (b, 0, n)),
            pl.BlockSpec((1, INS_IN, NBLK), lambda b, n: (b, 0, n)),
            pl.BlockSpec((INS_IN, SEM_IN), lambda b, n: (0, 0)),
            pl.BlockSpec((INS_IN, 1), lambda b, n: (0, 0)),
            pl.BlockSpec((INS_OUT, INS_IN), lambda b, n: (0, 0)),
            pl.BlockSpec((INS_OUT, 1), lambda b, n: (0, 0)),
        ],
        out_specs=pl.BlockSpec((1, INS_OUT, NBLK), lambda b, n: (b, 0, n)),
        out_shape=jax.ShapeDtypeStruct((B, INS_OUT, N), jnp.float32),
    )(f_sem, f_ins, W_eff, b_eff, W_ins, b_ins)


def _dist_body(p_ref, e_ref, d_ref, bm_ref, cnt_ref):
    p = p_ref[0]
    e = e_ref[0]
    sq_r = jnp.sum(p * p, axis=1, keepdims=True)
    sq_c = jnp.sum(e * e, axis=0, keepdims=True)
    dot = jax.lax.dot_general(p, e, (((1,), (0,)), ((), ())),
                              preferred_element_type=jnp.float32)
    d = (sq_r - 2.0 * dot) + sq_c
    d_ref[0] = d
    bm = jnp.min(d.reshape(DRB, NBM, N // NBM), axis=-1)
    t = jnp.max(bm, axis=-1, keepdims=True)
    bm_ref[0] = jnp.broadcast_to(t, (DRB, NBM))
    m01 = jnp.where(d <= t, 1.0, 0.0)
    cnt_ref[0] = jnp.sum(m01.reshape(DRB, NCH, 16), axis=-1)


def _dist(p, e_ins):
    return pl.pallas_call(
        _dist_body,
        grid=(B, N // DRB),
        in_specs=[
            pl.BlockSpec((1, DRB, INS_OUT), lambda b, n: (b, n, 0)),
            pl.BlockSpec((1, INS_OUT, N), lambda b, n: (b, 0, 0)),
        ],
        out_specs=[
            pl.BlockSpec((1, DRB, N), lambda b, n: (b, n, 0)),
            pl.BlockSpec((1, DRB, NBM), lambda b, n: (b, n, 0)),
            pl.BlockSpec((1, DRB, NCH), lambda b, n: (b, n, 0)),
        ],
        out_shape=[
            jax.ShapeDtypeStruct((B, N, N), jnp.float32),
            jax.ShapeDtypeStruct((B, N, NBM), jnp.float32),
            jax.ShapeDtypeStruct((B, N, NCH), jnp.float32),
        ],
    )(p, e_ins)


def _gperm(x, idx):
    dnums = lax.GatherDimensionNumbers(
        offset_dims=(), collapsed_slice_dims=(0,), start_index_map=(0,))
    return lax.gather(x, idx[:, None], dnums, (1,),
                      mode=lax.GatherScatterMode.PROMISE_IN_BOUNDS)


def _tree_min(v, perms):
    for p in perms:
        v = jnp.minimum(v, _gperm(v, p))
    return v


def _tree_sum(v, perms):
    for p in perms:
        v = v + _gperm(v, p)
    return v


def _cmpx(kv, ki, pv, pi, keepmin):
    lt = kv < pv
    mn = jnp.minimum(kv, pv)
    mx = jnp.maximum(kv, pv)
    mni = jnp.where(lt, ki, pi)
    mxi = jnp.where(lt, pi, ki)
    return jnp.where(keepmin, mn, mx), jnp.where(keepmin, mni, mxi)


def _km(it, ksz, j):
    partner = it ^ j
    ltp = jnp.where(it < partner, 1, 0)
    if ksz >= 16:
        return ltp > 0
    up = jnp.where((it & ksz) == 0, 1, 0)
    return ltp == up


def _sort16(kv, ki, it):
    for ksz in (2, 4, 8, 16):
        j = ksz >> 1
        while j >= 1:
            perm = it ^ j
            keepmin = _km(it, ksz, j)
            pk = _gperm(kv, perm)
            pi = _gperm(ki, perm)
            kv, ki = _cmpx(kv, ki, pk, pi, keepmin)
            j >>= 1
    return kv, ki


def _clean16(kv, ki, it):
    for j in (8, 4, 2, 1):
        perm = it ^ j
        keepmin = _km(it, 16, j)
        pk = _gperm(kv, perm)
        pi = _gperm(ki, perm)
        kv, ki = _cmpx(kv, ki, pk, pi, keepmin)
    return kv, ki


def _merge32_low(alov, aloi, ahiv, ahii, bv, bi, it):
    rbv = lax.rev(bv, (0,))
    rbi = lax.rev(bi, (0,))
    lt = ahiv < rbv
    l1v = jnp.minimum(ahiv, rbv)
    l1i = jnp.where(lt, ahii, rbi)
    lt2 = alov < l1v
    lov = jnp.minimum(alov, l1v)
    hiv = jnp.maximum(alov, l1v)
    loi = jnp.where(lt2, aloi, l1i)
    hii = jnp.where(lt2, l1i, aloi)
    lov, loi = _clean16(lov, loi, it)
    hiv, hii = _clean16(hiv, hii, it)
    return lov, loi, hiv, hii


def _sc_knn_pool(d_hbm, bm_hbm, cnt_hbm, fsemT_hbm, out_hbm,
                 row_buf, bm_buf, cnt_buf, cand_val, cand_idx, idx_buf, gbuf,
                 ostage, dsem0, dsem1, gsem0, gsem1):
    nc = 2
    wid = lax.axis_index("s") * nc + lax.axis_index("c")
    r0 = wid * ROWS_W
    boff = (wid // 8) * N
    it = lax.iota(jnp.int32, 16)
    inf = jnp.full((16,), jnp.inf, jnp.float32)
    zero_i = jnp.zeros((16,), jnp.int32)
    perms = tuple(it ^ sh for sh in (1, 2, 4, 8))
    dsems = (dsem0, dsem1)
    gsems = (gsem0, gsem1)

    def d_dma(slot, grp):
        base = r0 + grp * G
        return pltpu.make_async_copy(d_hbm.at[pl.ds(base, G)],
                                     row_buf.at[slot], dsems[slot])

    def bm_dma(slot, grp):
        base = r0 + grp * G
        return pltpu.make_async_copy(bm_hbm.at[pl.ds(base, G)],
                                     bm_buf.at[slot], dsems[slot])

    def cnt_dma(slot, grp):
        base = r0 + grp * G
        return pltpu.make_async_copy(cnt_hbm.at[pl.ds(base, G)],
                                     cnt_buf.at[slot], dsems[slot])

    def g_dma(slot, kk):
        return pltpu.make_async_copy(fsemT_hbm.at[idx_buf.at[slot, kk]],
                                     gbuf.at[slot, kk], gsems[slot])

    for s in (0, 1):
        d_dma(s, s).start()
        bm_dma(s, s).start()
        cnt_dma(s, s).start()

    def process_row(s, kk):
        def scan_group(jg, off):
            cv = cnt_buf[s, kk, pl.ds(jg * 16, 16)]
            for L in range(16):
                cnt = lax.convert_element_type(cv[L], jnp.int32)
                jbase = jnp.full((16,), (jg * 16 + L) * 16, jnp.int32)
                v0 = row_buf[s, kk, pl.ds((jg * 16 + L) * 16, 16)]

                def extract_body(_, carry, jbase=jbase):
                    vv, ofs = carry
                    mn = _tree_min(vv, perms)
                    eq = vv == mn
                    lanev = jnp.where(eq, it, 16)
                    lane = _tree_min(lanev, perms)
                    cand_val[pl.ds(ofs, 16)] = mn
                    cand_idx[pl.ds(ofs, 16)] = lane + jbase
                    vv = jnp.where(it == lane, jnp.inf, vv)
                    return (vv, ofs + 1)

                _, off = lax.fori_loop(0, cnt, extract_body, (v0, off))
            return off

        c = lax.fori_loop(0, NCH // 16, scan_group, 0)

        c_spl = jnp.full((16,), c, jnp.int32)
        nch = (c + 15) // 16

        def sel_body(i, carry):
            alov, aloi, ahiv, ahii = carry
            cv = cand_val[pl.ds(i * 16, 16)]
            ci = cand_idx[pl.ds(i * 16, 16)]
            valid = (it + jnp.full((16,), i * 16, jnp.int32)) < c_spl
            cv = jnp.where(valid, cv, jnp.inf)
            sv, si = _sort16(cv, ci, it)
            return _merge32_low(alov, aloi, ahiv, ahii, sv, si, it)

        _, loi, _, hii = lax.fori_loop(0, nch, sel_body,
                                       (inf, zero_i, inf, zero_i))
        bo = jnp.full((16,), boff, jnp.int32)
        idx_buf[s, kk, pl.ds(0, 16)] = loi + bo
        idx_buf[s, kk, pl.ds(16, 16)] = hii + bo
        g_dma(s, kk).start()

    def max_rows(s, grp):
        for kk in range(G):
            g_dma(s, kk).wait()
        for kk in range(G):
            def max_body(r, accs, kk=kk):
                return tuple(
                    jnp.maximum(accs[cc], gbuf[s, kk, r, pl.ds(16 * cc, 16)])
                    for cc in range(8))

            accs = tuple(gbuf[s, kk, 0, pl.ds(16 * cc, 16)] for cc in range(8))
            accs = lax.fori_loop(1, K, max_body, accs)
            for cc in range(8):
                ostage[kk, pl.ds(16 * cc, 16)] = accs[cc]
        base = r0 + grp * G
        pltpu.sync_copy(ostage, out_hbm.at[pl.ds(base, G)])

    def body(gi, _):
        for s in (0, 1):
            grp = 2 * gi + s
            d_dma(s, grp).wait()
            bm_dma(s, grp).wait()
            cnt_dma(s, grp).wait()
            for kk in range(G):
                process_row(s, kk)
            nxt = jnp.minimum(grp + 2, ROWS_W // G - 1)
            base_n = r0 + nxt * G
            pltpu.make_async_copy(d_hbm.at[pl.ds(base_n, G)],
                                  row_buf.at[s], dsems[s]).start()
            pltpu.make_async_copy(bm_hbm.at[pl.ds(base_n, G)],
                                  bm_buf.at[s], dsems[s]).start()
            pltpu.make_async_copy(cnt_hbm.at[pl.ds(base_n, G)],
                                  cnt_buf.at[s], dsems[s]).start()

            @pl.when(grp > 0)
            def _():
                max_rows(1 - s, grp - 1)
        return 0

    lax.fori_loop(0, NGRP, body, 0)
    max_rows(1, ROWS_W // G - 1)
    for s in (0, 1):
        d_dma(s, 0).wait()
        bm_dma(s, 0).wait()
        cnt_dma(s, 0).wait()


def _sc_knn(d2, bm2, cnt2, fsemT):
    mesh = plsc.VectorSubcoreMesh(core_axis_name="c", subcore_axis_name="s")
    call = pl.kernel(
        _sc_knn_pool, mesh=mesh,
        out_type=jax.ShapeDtypeStruct((B * N, SEM_IN), jnp.float32),
        scratch_types=[
            pltpu.VMEM((2, G, N), jnp.float32),
            pltpu.VMEM((2, G, NBM), jnp.float32),
            pltpu.VMEM((2, G, NCH), jnp.float32),
            pltpu.VMEM((CAP + 16,), jnp.float32),
            pltpu.VMEM((CAP + 16,), jnp.int32),
            pltpu.VMEM((2, G, K), jnp.int32),
            pltpu.VMEM((2, G, K, SEM_IN), jnp.float32),
            pltpu.VMEM((G, SEM_IN), jnp.float32),
            pltpu.SemaphoreType.DMA,
            pltpu.SemaphoreType.DMA,
            pltpu.SemaphoreType.DMA,
            pltpu.SemaphoreType.DMA,
        ],
    )
    return call(d2, bm2, cnt2, fsemT)


def _proj_body(fisem_ref, W_ref, b_ref, out_ref):
    out_ref[...] = jax.lax.dot_general(
        fisem_ref[...], W_ref[...], (((1,), (0,)), ((), ())),
        preferred_element_type=jnp.float32) + b_ref[...]


def _proj(f_isemT, W_semT, b_sem):
    return pl.pallas_call(
        _proj_body,
        grid=((B * N) // NBLK,),
        in_specs=[
            pl.BlockSpec((NBLK, SEM_IN), lambda n: (n, 0)),
            pl.BlockSpec((SEM_IN, SEM_OUT), lambda n: (0, 0)),
            pl.BlockSpec((1, SEM_OUT), lambda n: (0, 0)),
        ],
        out_specs=pl.BlockSpec((NBLK, SEM_OUT), lambda n: (n, 0)),
        out_shape=jax.ShapeDtypeStruct((B * N, SEM_OUT), jnp.float32),
    )(f_isemT, W_semT, b_sem)


def kernel(f_sem, f_ins, W_ad, b_ad, gamma_ad, beta_ad, W_ins, b_ins, W_sem, b_sem):
    W_eff = gamma_ad[:, None] * W_ad
    b_eff = (gamma_ad * b_ad + beta_ad)[:, None]
    e_ins = _dense_front(f_sem, f_ins, W_eff, b_eff, W_ins, b_ins[:, None])

    p = jnp.transpose(e_ins, (0, 2, 1))
    d, bm, cnt = _dist(p, e_ins)

    fsemT = jnp.transpose(f_sem, (0, 2, 1)).reshape(B * N, SEM_IN)
    f_isemT = _sc_knn(d.reshape(B * N, N), bm.reshape(B * N, NBM),
                      cnt.reshape(B * N, NCH), fsemT)

    p_semT = _proj(f_isemT, jnp.transpose(W_sem), b_sem[None, :])
    p_sem = jnp.transpose(p_semT.reshape(B, N, SEM_OUT), (0, 2, 1))
    return (p_sem, e_ins)


# speedup vs baseline: 3.8470x; 3.3868x over previous
import functools

import jax
import jax.numpy as jnp
import numpy as np
from jax import lax
from jax.experimental import pallas as pl
from jax.experimental.pallas import tpu as pltpu
from jax.experimental.pallas import tpu_sc as plsc

B, N = 4, 4096
SEM_IN, SEM_OUT = 128, 13
INS_IN, INS_OUT = 128, 32
K = 32

NBLK = 512
DRB = 512
NBM = 32
NCH = 256

NW = 32
ROWS_W = (B * N) // NW
G = 2
NGRP = ROWS_W // (2 * G)
CAP = N


def _dense_front_body(fsem_ref, fins_ref, Wad_ref, bad_ref, Wins_ref, bins_ref,
                      eins_ref):
    fsem = fsem_ref[0]
    fins = fins_ref[0]
    adapted = jnp.maximum(
        jax.lax.dot_general(Wad_ref[...], fsem, (((1,), (0,)), ((), ())),
                            preferred_element_type=jnp.float32) + bad_ref[...],
        0.0)
    f_sins = fins + adapted
    eins = jax.lax.dot_general(Wins_ref[...], f_sins, (((1,), (0,)), ((), ())),
                               preferred_element_type=jnp.float32) + bins_ref[...]
    eins_ref[0] = eins


def _dense_front(f_sem, f_ins, W_eff, b_eff, W_ins, b_ins):
    return pl.pallas_call(
        _dense_front_body,
        grid=(B, N // NBLK),
        in_specs=[
            pl.BlockSpec((1, SEM_IN, NBLK), lambda b, n: (b, 0, n)),
            pl.BlockSpec((1, INS_IN, NBLK), lambda b, n: (b, 0, n)),
            pl.BlockSpec((INS_IN, SEM_IN), lambda b, n: (0, 0)),
            pl.BlockSpec((INS_IN, 1), lambda b, n: (0, 0)),
            pl.BlockSpec((INS_OUT, INS_IN), lambda b, n: (0, 0)),
            pl.BlockSpec((INS_OUT, 1), lambda b, n: (0, 0)),
        ],
        out_specs=pl.BlockSpec((1, INS_OUT, NBLK), lambda b, n: (b, 0, n)),
        out_shape=jax.ShapeDtypeStruct((B, INS_OUT, N), jnp.float32),
    )(f_sem, f_ins, W_eff, b_eff, W_ins, b_ins)


def _dist_body(p_ref, e_ref, d_ref, bm_ref):
    p = p_ref[0]
    e = e_ref[0]
    sq_r = jnp.sum(p * p, axis=1, keepdims=True)
    sq_c = jnp.sum(e * e, axis=0, keepdims=True)
    dot = jax.lax.dot_general(p, e, (((1,), (0,)), ((), ())),
                              preferred_element_type=jnp.float32)
    d = (sq_r - 2.0 * dot) + sq_c
    d_ref[0] = d
    bm = jnp.min(d.reshape(DRB, NBM, N // NBM), axis=-1)
    bm_ref[0] = jnp.broadcast_to(jnp.max(bm, axis=-1, keepdims=True),
                                 (DRB, NBM))


def _dist(p, e_ins):
    return pl.pallas_call(
        _dist_body,
        grid=(B, N // DRB),
        in_specs=[
            pl.BlockSpec((1, DRB, INS_OUT), lambda b, n: (b, n, 0)),
            pl.BlockSpec((1, INS_OUT, N), lambda b, n: (b, 0, 0)),
        ],
        out_specs=[
            pl.BlockSpec((1, DRB, N), lambda b, n: (b, n, 0)),
            pl.BlockSpec((1, DRB, NBM), lambda b, n: (b, n, 0)),
        ],
        out_shape=[
            jax.ShapeDtypeStruct((B, N, N), jnp.float32),
            jax.ShapeDtypeStruct((B, N, NBM), jnp.float32),
        ],
    )(p, e_ins)


def _gperm(x, idx):
    dnums = lax.GatherDimensionNumbers(
        offset_dims=(), collapsed_slice_dims=(0,), start_index_map=(0,))
    return lax.gather(x, idx[:, None], dnums, (1,),
                      mode=lax.GatherScatterMode.PROMISE_IN_BOUNDS)


def _tree_min(v, perms):
    for p in perms:
        v = jnp.minimum(v, _gperm(v, p))
    return v


def _tree_sum(v, perms):
    for p in perms:
        v = v + _gperm(v, p)
    return v


def _cmpx(kv, ki, pv, pi, keepmin):
    lt = kv < pv
    mn = jnp.minimum(kv, pv)
    mx = jnp.maximum(kv, pv)
    mni = jnp.where(lt, ki, pi)
    mxi = jnp.where(lt, pi, ki)
    return jnp.where(keepmin, mn, mx), jnp.where(keepmin, mni, mxi)


def _km(it, ksz, j):
    partner = it ^ j
    ltp = jnp.where(it < partner, 1, 0)
    if ksz >= 16:
        return ltp > 0
    up = jnp.where((it & ksz) == 0, 1, 0)
    return ltp == up


def _sort16(kv, ki, it):
    for ksz in (2, 4, 8, 16):
        j = ksz >> 1
        while j >= 1:
            perm = it ^ j
            keepmin = _km(it, ksz, j)
            pk = _gperm(kv, perm)
            pi = _gperm(ki, perm)
            kv, ki = _cmpx(kv, ki, pk, pi, keepmin)
            j >>= 1
    return kv, ki


def _clean16(kv, ki, it):
    for j in (8, 4, 2, 1):
        perm = it ^ j
        keepmin = _km(it, 16, j)
        pk = _gperm(kv, perm)
        pi = _gperm(ki, perm)
        kv, ki = _cmpx(kv, ki, pk, pi, keepmin)
    return kv, ki


def _merge32_low(alov, aloi, ahiv, ahii, bv, bi, it):
    rbv = lax.rev(bv, (0,))
    rbi = lax.rev(bi, (0,))
    lt = ahiv < rbv
    l1v = jnp.minimum(ahiv, rbv)
    l1i = jnp.where(lt, ahii, rbi)
    lt2 = alov < l1v
    lov = jnp.minimum(alov, l1v)
    hiv = jnp.maximum(alov, l1v)
    loi = jnp.where(lt2, aloi, l1i)
    hii = jnp.where(lt2, l1i, aloi)
    lov, loi = _clean16(lov, loi, it)
    hiv, hii = _clean16(hiv, hii, it)
    return lov, loi, hiv, hii


def _sc_knn_pool(d_hbm, bm_hbm, fsemT_hbm, out_hbm,
                 row_buf, bm_buf, cand_val, cand_idx, idx_buf, gbuf,
                 ostage, dsem0, dsem1, gsem0, gsem1):
    nc = 2
    wid = lax.axis_index("s") * nc + lax.axis_index("c")
    r0 = wid * ROWS_W
    boff = (wid // 8) * N
    it = lax.iota(jnp.int32, 16)
    inf = jnp.full((16,), jnp.inf, jnp.float32)
    zero_i = jnp.zeros((16,), jnp.int32)
    perms = tuple(it ^ sh for sh in (1, 2, 4, 8))
    dsems = (dsem0, dsem1)
    gsems = (gsem0, gsem1)

    def d_dma(slot, grp):
        base = r0 + grp * G
        return pltpu.make_async_copy(d_hbm.at[pl.ds(base, G)],
                                     row_buf.at[slot], dsems[slot])

    def bm_dma(slot, grp):
        base = r0 + grp * G
        return pltpu.make_async_copy(bm_hbm.at[pl.ds(base, G)],
                                     bm_buf.at[slot], dsems[slot])

    def g_dma(slot, kk):
        return pltpu.make_async_copy(fsemT_hbm.at[idx_buf.at[slot, kk]],
                                     gbuf.at[slot, kk], gsems[slot])

    for s in (0, 1):
        d_dma(s, s).start()
        bm_dma(s, s).start()

    def process_row(s, kk):
        t_spl = bm_buf[s, kk, pl.ds(0, 16)]

        def scan_body(j, off):
            v = row_buf[s, kk, pl.ds(j * 16, 16)]
            m01 = jnp.where(v <= t_spl, 1, 0)
            cnt = _tree_sum(m01, perms)[0]
            jbase = jnp.full((16,), j * 16, jnp.int32)

            def extract_body(_, carry):
                vv, ofs = carry
                mn = _tree_min(vv, perms)
                eq = vv == mn
                lanev = jnp.where(eq, it, 16)
                lane = _tree_min(lanev, perms)
                cand_val[pl.ds(ofs, 16)] = mn
                cand_idx[pl.ds(ofs, 16)] = lane + jbase
                vv = jnp.where(it == lane, jnp.inf, vv)
                return (vv, ofs + 1)

            _, off = lax.fori_loop(0, cnt, extract_body, (v, off))
            return off

        c = lax.fori_loop(0, NCH, scan_body, 0)

        c_spl = jnp.full((16,), c, jnp.int32)
        nch = (c + 15) // 16

        def sel_body(i, carry):
            alov, aloi, ahiv, ahii = carry
            cv = cand_val[pl.ds(i * 16, 16)]
            ci = cand_idx[pl.ds(i * 16, 16)]
            valid = (it + jnp.full((16,), i * 16, jnp.int32)) < c_spl
            cv = jnp.where(valid, cv, jnp.inf)
            sv, si = _sort16(cv, ci, it)
            return _merge32_low(alov, aloi, ahiv, ahii, sv, si, it)

        _, loi, _, hii = lax.fori_loop(0, nch, sel_body,
                                       (inf, zero_i, inf, zero_i))
        bo = jnp.full((16,), boff, jnp.int32)
        idx_buf[s, kk, pl.ds(0, 16)] = loi + bo
        idx_buf[s, kk, pl.ds(16, 16)] = hii + bo
        g_dma(s, kk).start()

    def max_rows(s, grp):
        for kk in range(G):
            g_dma(s, kk).wait()
        for kk in range(G):
            def max_body(r, accs, kk=kk):
                return tuple(
                    jnp.maximum(accs[cc], gbuf[s, kk, r, pl.ds(16 * cc, 16)])
                    for cc in range(8))

            accs = tuple(gbuf[s, kk, 0, pl.ds(16 * cc, 16)] for cc in range(8))
            accs = lax.fori_loop(1, K, max_body, accs)
            for cc in range(8):
                ostage[kk, pl.ds(16 * cc, 16)] = accs[cc]
        base = r0 + grp * G
        pltpu.sync_copy(ostage, out_hbm.at[pl.ds(base, G)])

    def body(gi, _):
        for s in (0, 1):
            grp = 2 * gi + s
            d_dma(s, grp).wait()
            bm_dma(s, grp).wait()
            for kk in range(G):
                process_row(s, kk)
            nxt = jnp.minimum(grp + 2, ROWS_W // G - 1)
            base_n = r0 + nxt * G
            pltpu.make_async_copy(d_hbm.at[pl.ds(base_n, G)],
                                  row_buf.at[s], dsems[s]).start()
            pltpu.make_async_copy(bm_hbm.at[pl.ds(base_n, G)],
                                  bm_buf.at[s], dsems[s]).start()

            @pl.when(grp > 0)
            def _():
                max_rows(1 - s, grp - 1)
        return 0

    lax.fori_loop(0, NGRP, body, 0)
    max_rows(1, ROWS_W // G - 1)
    for s in (0, 1):
        d_dma(s, 0).wait()
        bm_dma(s, 0).wait()


def _sc_knn(d2, bm2, fsemT):
    mesh = plsc.VectorSubcoreMesh(core_axis_name="c", subcore_axis_name="s")
    call = pl.kernel(
        _sc_knn_pool, mesh=mesh,
        out_type=jax.ShapeDtypeStruct((B * N, SEM_IN), jnp.float32),
        scratch_types=[
            pltpu.VMEM((2, G, N), jnp.float32),
            pltpu.VMEM((2, G, NBM), jnp.float32),
            pltpu.VMEM((CAP + 16,), jnp.float32),
            pltpu.VMEM((CAP + 16,), jnp.int32),
            pltpu.VMEM((2, G, K), jnp.int32),
            pltpu.VMEM((2, G, K, SEM_IN), jnp.float32),
            pltpu.VMEM((G, SEM_IN), jnp.float32),
            pltpu.SemaphoreType.DMA,
            pltpu.SemaphoreType.DMA,
            pltpu.SemaphoreType.DMA,
            pltpu.SemaphoreType.DMA,
        ],
    )
    return call(d2, bm2, fsemT)


def _proj_body(fisem_ref, W_ref, b_ref, out_ref):
    out_ref[...] = jax.lax.dot_general(
        fisem_ref[...], W_ref[...], (((1,), (0,)), ((), ())),
        preferred_element_type=jnp.float32) + b_ref[...]


def _proj(f_isemT, W_semT, b_sem):
    return pl.pallas_call(
        _proj_body,
        grid=((B * N) // NBLK,),
        in_specs=[
            pl.BlockSpec((NBLK, SEM_IN), lambda n: (n, 0)),
            pl.BlockSpec((SEM_IN, SEM_OUT), lambda n: (0, 0)),
            pl.BlockSpec((1, SEM_OUT), lambda n: (0, 0)),
        ],
        out_specs=pl.BlockSpec((NBLK, SEM_OUT), lambda n: (n, 0)),
        out_shape=jax.ShapeDtypeStruct((B * N, SEM_OUT), jnp.float32),
    )(f_isemT, W_semT, b_sem)


def kernel(f_sem, f_ins, W_ad, b_ad, gamma_ad, beta_ad, W_ins, b_ins, W_sem, b_sem):
    W_eff = gamma_ad[:, None] * W_ad
    b_eff = (gamma_ad * b_ad + beta_ad)[:, None]
    e_ins = _dense_front(f_sem, f_ins, W_eff, b_eff, W_ins, b_ins[:, None])

    p = jnp.transpose(e_ins, (0, 2, 1))
    d, bm = _dist(p, e_ins)

    fsemT = jnp.transpose(f_sem, (0, 2, 1)).reshape(B * N, SEM_IN)
    f_isemT = _sc_knn(d.reshape(B * N, N), bm.reshape(B * N, NBM), fsemT)

    p_semT = _proj(f_isemT, jnp.transpose(W_sem), b_sem[None, :])
    p_sem = jnp.transpose(p_semT.reshape(B, N, SEM_OUT), (0, 2, 1))
    return (p_sem, e_ins)


# speedup vs baseline: 4.0585x; 1.0550x over previous
import functools

import jax
import jax.numpy as jnp
import numpy as np
from jax import lax
from jax.experimental import pallas as pl
from jax.experimental.pallas import tpu as pltpu
from jax.experimental.pallas import tpu_sc as plsc

B, N = 4, 4096
SEM_IN, SEM_OUT = 128, 13
INS_IN, INS_OUT = 128, 32
K = 32

NBLK = 512
DRB = 512
NBM = 32
NCH = 256

NW = 32
ROWS_W = (B * N) // NW
G = 4
NGRP = ROWS_W // (2 * G)
CAP = N


def _dense_front_body(fsem_ref, fins_ref, Wad_ref, bad_ref, Wins_ref, bins_ref,
                      eins_ref):
    fsem = fsem_ref[0]
    fins = fins_ref[0]
    adapted = jnp.maximum(
        jax.lax.dot_general(Wad_ref[...], fsem, (((1,), (0,)), ((), ())),
                            preferred_element_type=jnp.float32) + bad_ref[...],
        0.0)
    f_sins = fins + adapted
    eins = jax.lax.dot_general(Wins_ref[...], f_sins, (((1,), (0,)), ((), ())),
                               preferred_element_type=jnp.float32) + bins_ref[...]
    eins_ref[0] = eins


def _dense_front(f_sem, f_ins, W_eff, b_eff, W_ins, b_ins):
    return pl.pallas_call(
        _dense_front_body,
        grid=(B, N // NBLK),
        in_specs=[
            pl.BlockSpec((1, SEM_IN, NBLK), lambda b, n: (b, 0, n)),
            pl.BlockSpec((1, INS_IN, NBLK), lambda b, n: (b, 0, n)),
            pl.BlockSpec((INS_IN, SEM_IN), lambda b, n: (0, 0)),
            pl.BlockSpec((INS_IN, 1), lambda b, n: (0, 0)),
            pl.BlockSpec((INS_OUT, INS_IN), lambda b, n: (0, 0)),
            pl.BlockSpec((INS_OUT, 1), lambda b, n: (0, 0)),
        ],
        out_specs=pl.BlockSpec((1, INS_OUT, NBLK), lambda b, n: (b, 0, n)),
        out_shape=jax.ShapeDtypeStruct((B, INS_OUT, N), jnp.float32),
    )(f_sem, f_ins, W_eff, b_eff, W_ins, b_ins)


def _dist_body(p_ref, e_ref, d_ref, bm_ref):
    p = p_ref[0]
    e = e_ref[0]
    sq_r = jnp.sum(p * p, axis=1, keepdims=True)
    sq_c = jnp.sum(e * e, axis=0, keepdims=True)
    dot = jax.lax.dot_general(p, e, (((1,), (0,)), ((), ())),
                              preferred_element_type=jnp.float32)
    d = (sq_r - 2.0 * dot) + sq_c
    d_ref[0] = d
    bm = jnp.min(d.reshape(DRB, NBM, N // NBM), axis=-1)
    bm_ref[0] = jnp.broadcast_to(jnp.max(bm, axis=-1, keepdims=True),
                                 (DRB, NBM))


def _dist(p, e_ins):
    return pl.pallas_call(
        _dist_body,
        grid=(B, N // DRB),
        in_specs=[
            pl.BlockSpec((1, DRB, INS_OUT), lambda b, n: (b, n, 0)),
            pl.BlockSpec((1, INS_OUT, N), lambda b, n: (b, 0, 0)),
        ],
        out_specs=[
            pl.BlockSpec((1, DRB, N), lambda b, n: (b, n, 0)),
            pl.BlockSpec((1, DRB, NBM), lambda b, n: (b, n, 0)),
        ],
        out_shape=[
            jax.ShapeDtypeStruct((B, N, N), jnp.float32),
            jax.ShapeDtypeStruct((B, N, NBM), jnp.float32),
        ],
    )(p, e_ins)


def _gperm(x, idx):
    dnums = lax.GatherDimensionNumbers(
        offset_dims=(), collapsed_slice_dims=(0,), start_index_map=(0,))
    return lax.gather(x, idx[:, None], dnums, (1,),
                      mode=lax.GatherScatterMode.PROMISE_IN_BOUNDS)


def _tree_min(v, perms):
    for p in perms:
        v = jnp.minimum(v, _gperm(v, p))
    return v


def _tree_sum(v, perms):
    for p in perms:
        v = v + _gperm(v, p)
    return v


def _cmpx(kv, ki, pv, pi, keepmin):
    lt = kv < pv
    mn = jnp.minimum(kv, pv)
    mx = jnp.maximum(kv, pv)
    mni = jnp.where(lt, ki, pi)
    mxi = jnp.where(lt, pi, ki)
    return jnp.where(keepmin, mn, mx), jnp.where(keepmin, mni, mxi)


def _km(it, ksz, j):
    partner = it ^ j
    ltp = jnp.where(it < partner, 1, 0)
    if ksz >= 16:
        return ltp > 0
    up = jnp.where((it & ksz) == 0, 1, 0)
    return ltp == up


def _sort16(kv, ki, it):
    for ksz in (2, 4, 8, 16):
        j = ksz >> 1
        while j >= 1:
            perm = it ^ j
            keepmin = _km(it, ksz, j)
            pk = _gperm(kv, perm)
            pi = _gperm(ki, perm)
            kv, ki = _cmpx(kv, ki, pk, pi, keepmin)
            j >>= 1
    return kv, ki


def _clean16(kv, ki, it):
    for j in (8, 4, 2, 1):
        perm = it ^ j
        keepmin = _km(it, 16, j)
        pk = _gperm(kv, perm)
        pi = _gperm(ki, perm)
        kv, ki = _cmpx(kv, ki, pk, pi, keepmin)
    return kv, ki


def _merge32_low(alov, aloi, ahiv, ahii, bv, bi, it):
    rbv = lax.rev(bv, (0,))
    rbi = lax.rev(bi, (0,))
    lt = ahiv < rbv
    l1v = jnp.minimum(ahiv, rbv)
    l1i = jnp.where(lt, ahii, rbi)
    lt2 = alov < l1v
    lov = jnp.minimum(alov, l1v)
    hiv = jnp.maximum(alov, l1v)
    loi = jnp.where(lt2, aloi, l1i)
    hii = jnp.where(lt2, l1i, aloi)
    lov, loi = _clean16(lov, loi, it)
    hiv, hii = _clean16(hiv, hii, it)
    return lov, loi, hiv, hii


def _sc_knn_pool(d_hbm, bm_hbm, fsemT_hbm, out_hbm,
                 row_buf, bm_buf, cand_val, cand_idx, idx_buf, gbuf,
                 ostage, dsem0, dsem1, gsem0, gsem1):
    nc = 2
    wid = lax.axis_index("s") * nc + lax.axis_index("c")
    r0 = wid * ROWS_W
    boff = (wid // 8) * N
    it = lax.iota(jnp.int32, 16)
    inf = jnp.full((16,), jnp.inf, jnp.float32)
    zero_i = jnp.zeros((16,), jnp.int32)
    perms = tuple(it ^ sh for sh in (1, 2, 4, 8))
    dsems = (dsem0, dsem1)
    gsems = (gsem0, gsem1)

    def d_dma(slot, grp):
        base = r0 + grp * G
        return pltpu.make_async_copy(d_hbm.at[pl.ds(base, G)],
                                     row_buf.at[slot], dsems[slot])

    def bm_dma(slot, grp):
        base = r0 + grp * G
        return pltpu.make_async_copy(bm_hbm.at[pl.ds(base, G)],
                                     bm_buf.at[slot], dsems[slot])

    def g_dma(slot, kk):
        return pltpu.make_async_copy(fsemT_hbm.at[idx_buf.at[slot, kk]],
                                     gbuf.at[slot, kk], gsems[slot])

    for s in (0, 1):
        d_dma(s, s).start()
        bm_dma(s, s).start()

    def process_row(s, kk):
        t_spl = bm_buf[s, kk, pl.ds(0, 16)]

        def scan_body(j2, off):
            for half in (0, 1):
                j16 = (j2 * 2 + half) * 16
                v = row_buf[s, kk, pl.ds(j16, 16)]
                m01 = jnp.where(v <= t_spl, 1, 0)
                cnt = _tree_sum(m01, perms)[0]
                jbase = jnp.full((16,), j16, jnp.int32)

                def extract_body(_, carry, jbase=jbase):
                    vv, ofs = carry
                    mn = _tree_min(vv, perms)
                    eq = vv == mn
                    lanev = jnp.where(eq, it, 16)
                    lane = _tree_min(lanev, perms)
                    cand_val[pl.ds(ofs, 16)] = mn
                    cand_idx[pl.ds(ofs, 16)] = lane + jbase
                    vv = jnp.where(it == lane, jnp.inf, vv)
                    return (vv, ofs + 1)

                _, off = lax.fori_loop(0, cnt, extract_body, (v, off))
            return off

        c = lax.fori_loop(0, NCH // 2, scan_body, 0)

        c_spl = jnp.full((16,), c, jnp.int32)
        nch = (c + 15) // 16

        def sel_body(i, carry):
            alov, aloi, ahiv, ahii = carry
            cv = cand_val[pl.ds(i * 16, 16)]
            ci = cand_idx[pl.ds(i * 16, 16)]
            valid = (it + jnp.full((16,), i * 16, jnp.int32)) < c_spl
            cv = jnp.where(valid, cv, jnp.inf)
            sv, si = _sort16(cv, ci, it)
            return _merge32_low(alov, aloi, ahiv, ahii, sv, si, it)

        _, loi, _, hii = lax.fori_loop(0, nch, sel_body,
                                       (inf, zero_i, inf, zero_i))
        bo = jnp.full((16,), boff, jnp.int32)
        idx_buf[s, kk, pl.ds(0, 16)] = loi + bo
        idx_buf[s, kk, pl.ds(16, 16)] = hii + bo
        g_dma(s, kk).start()

    def max_rows(s, grp):
        for kk in range(G):
            g_dma(s, kk).wait()
        for kk in range(G):
            def max_body(r, accs, kk=kk):
                return tuple(
                    jnp.maximum(accs[cc], gbuf[s, kk, r, pl.ds(16 * cc, 16)])
                    for cc in range(8))

            accs = tuple(gbuf[s, kk, 0, pl.ds(16 * cc, 16)] for cc in range(8))
            accs = lax.fori_loop(1, K, max_body, accs)
            for cc in range(8):
                ostage[kk, pl.ds(16 * cc, 16)] = accs[cc]
        base = r0 + grp * G
        pltpu.sync_copy(ostage, out_hbm.at[pl.ds(base, G)])

    def body(gi, _):
        for s in (0, 1):
            grp = 2 * gi + s
            d_dma(s, grp).wait()
            bm_dma(s, grp).wait()
            for kk in range(G):
                process_row(s, kk)
            nxt = jnp.minimum(grp + 2, ROWS_W // G - 1)
            base_n = r0 + nxt * G
            pltpu.make_async_copy(d_hbm.at[pl.ds(base_n, G)],
                                  row_buf.at[s], dsems[s]).start()
            pltpu.make_async_copy(bm_hbm.at[pl.ds(base_n, G)],
                                  bm_buf.at[s], dsems[s]).start()

            @pl.when(grp > 0)
            def _():
                max_rows(1 - s, grp - 1)
        return 0

    lax.fori_loop(0, NGRP, body, 0)
    max_rows(1, ROWS_W // G - 1)
    for s in (0, 1):
        d_dma(s, 0).wait()
        bm_dma(s, 0).wait()


def _sc_knn(d2, bm2, fsemT):
    mesh = plsc.VectorSubcoreMesh(core_axis_name="c", subcore_axis_name="s")
    call = pl.kernel(
        _sc_knn_pool, mesh=mesh,
        out_type=jax.ShapeDtypeStruct((B * N, SEM_IN), jnp.float32),
        scratch_types=[
            pltpu.VMEM((2, G, N), jnp.float32),
            pltpu.VMEM((2, G, NBM), jnp.float32),
            pltpu.VMEM((CAP + 16,), jnp.float32),
            pltpu.VMEM((CAP + 16,), jnp.int32),
            pltpu.VMEM((2, G, K), jnp.int32),
            pltpu.VMEM((2, G, K, SEM_IN), jnp.float32),
            pltpu.VMEM((G, SEM_IN), jnp.float32),
            pltpu.SemaphoreType.DMA,
            pltpu.SemaphoreType.DMA,
            pltpu.SemaphoreType.DMA,
            pltpu.SemaphoreType.DMA,
        ],
    )
    return call(d2, bm2, fsemT)


def _proj_body(fisem_ref, W_ref, b_ref, out_ref):
    out_ref[...] = jax.lax.dot_general(
        fisem_ref[...], W_ref[...], (((1,), (0,)), ((), ())),
        preferred_element_type=jnp.float32) + b_ref[...]


def _proj(f_isemT, W_semT, b_sem):
    return pl.pallas_call(
        _proj_body,
        grid=((B * N) // NBLK,),
        in_specs=[
            pl.BlockSpec((NBLK, SEM_IN), lambda n: (n, 0)),
            pl.BlockSpec((SEM_IN, SEM_OUT), lambda n: (0, 0)),
            pl.BlockSpec((1, SEM_OUT), lambda n: (0, 0)),
        ],
        out_specs=pl.BlockSpec((NBLK, SEM_OUT), lambda n: (n, 0)),
        out_shape=jax.ShapeDtypeStruct((B * N, SEM_OUT), jnp.float32),
    )(f_isemT, W_semT, b_sem)


def kernel(f_sem, f_ins, W_ad, b_ad, gamma_ad, beta_ad, W_ins, b_ins, W_sem, b_sem):
    W_eff = gamma_ad[:, None] * W_ad
    b_eff = (gamma_ad * b_ad + beta_ad)[:, None]
    e_ins = _dense_front(f_sem, f_ins, W_eff, b_eff, W_ins, b_ins[:, None])

    p = jnp.transpose(e_ins, (0, 2, 1))
    d, bm = _dist(p, e_ins)

    fsemT = jnp.transpose(f_sem, (0, 2, 1)).reshape(B * N, SEM_IN)
    f_isemT = _sc_knn(d.reshape(B * N, N), bm.reshape(B * N, NBM), fsemT)

    p_semT = _proj(f_isemT, jnp.transpose(W_sem), b_sem[None, :])
    p_sem = jnp.transpose(p_semT.reshape(B, N, SEM_OUT), (0, 2, 1))
    return (p_sem, e_ins)
